# Initial kernel scaffold; baseline (speedup 1.0000x reference)
import jax
import jax.numpy as jnp
from jax.experimental import pallas as pl


def kernel(feats, W, b):
    raise NotImplementedError("write your pallas kernel here")


# speedup vs baseline: 3.8366x; 3.8366x over previous
import functools

import jax
import jax.numpy as jnp
from jax import lax
from jax.experimental import pallas as pl
from jax.experimental.pallas import tpu as pltpu
from jax.experimental.pallas import tpu_sc as plsc

N = 32768
D = 512
LANES = 16
NC = 2
NS = 16
NW = NC * NS
RPW = N // NW
CH = 64
NCHUNK = RPW // CH
KV = D // LANES

_mesh = plsc.VectorSubcoreMesh(core_axis_name="c", subcore_axis_name="s")


@functools.partial(
    pl.kernel,
    out_type=(
        jax.ShapeDtypeStruct((NW, LANES), jnp.float32),
        jax.ShapeDtypeStruct((NW, LANES), jnp.int32),
    ),
    mesh=_mesh,
    compiler_params=pltpu.CompilerParams(needs_layout_passes=False),
    scratch_types=(
        pltpu.VMEM((D,), jnp.float32),
        pltpu.VMEM((2, CH, D), jnp.float32),
        pltpu.VMEM((1, LANES), jnp.float32),
        pltpu.VMEM((1, LANES), jnp.int32),
        pltpu.SemaphoreType.DMA,
        pltpu.SemaphoreType.DMA,
    ),
)
def _scan_kernel(feats_hbm, wd_hbm, vals_out, idx_out, wd_v, buf, rec_v,
                 rec_i, sem0, sem1):
    ci = lax.axis_index("c")
    si = lax.axis_index("s")
    wid = si * NC + ci
    base = wid * RPW

    pltpu.sync_copy(wd_hbm, wd_v)
    wv = [wd_v[pl.ds(LANES * k, LANES)] for k in range(KV)]

    sems = (sem0, sem1)

    def start(c):
        slot = c % 2
        return pltpu.async_copy(
            feats_hbm.at[pl.ds(base + c * CH, CH), :], buf.at[slot],
            sems[slot])

    handles = {0: start(0), 1: start(1)}

    carry = (jnp.float32(-jnp.inf), jnp.int32(0),
             jnp.float32(jnp.inf), jnp.int32(0))

    for c in range(NCHUNK):
        slot = c % 2
        handles[c].wait()
        cbase = base + c * CH

        def row_body(r, cr, slot=slot, cbase=cbase):
            bmaxv, bmaxi, bminv, bmini = cr
            acc = [buf[slot, r, pl.ds(LANES * k, LANES)] * wv[k]
                   for k in range(4)]
            for k in range(4, KV):
                acc[k % 4] = acc[k % 4] + (
                    buf[slot, r, pl.ds(LANES * k, LANES)] * wv[k])
            s = (acc[0] + acc[1]) + (acc[2] + acc[3])
            d = jnp.sum(s)
            ridx = (cbase + r).astype(jnp.int32)
            upmax = d > bmaxv
            bmaxv = jnp.where(upmax, d, bmaxv)
            bmaxi = jnp.where(upmax, ridx, bmaxi)
            upmin = d < bminv
            bminv = jnp.where(upmin, d, bminv)
            bmini = jnp.where(upmin, ridx, bmini)
            return (bmaxv, bmaxi, bminv, bmini)

        carry = lax.fori_loop(0, CH, row_body, carry)
        if c + 2 < NCHUNK:
            handles[c + 2] = start(c + 2)

    bmaxv, bmaxi, bminv, bmini = carry
    lane = lax.iota(jnp.int32, LANES)
    rec_v[0] = jnp.where(lane == 0, bmaxv,
                         jnp.where(lane == 1, bminv,
                                   jnp.zeros((LANES,), jnp.float32)))
    rec_i[0] = jnp.where(lane == 0, bmaxi,
                         jnp.where(lane == 1, bmini,
                                   jnp.zeros((LANES,), jnp.int32)))
    pltpu.sync_copy(rec_v, vals_out.at[pl.ds(wid, 1)])
    pltpu.sync_copy(rec_i, idx_out.at[pl.ds(wid, 1)])


def _merge_body(cand_v, cand_i, feats, out, sem0, sem1):
    bmaxv = cand_v[0, 0]
    bmaxi = cand_i[0, 0]
    bminv = cand_v[0, 1]
    bmini = cand_i[0, 1]
    for w in range(1, NW):
        v0 = cand_v[w, 0]
        i0 = cand_i[w, 0]
        t0 = (v0 > bmaxv) | ((v0 == bmaxv) & (i0 < bmaxi))
        bmaxv = jnp.where(t0, v0, bmaxv)
        bmaxi = jnp.where(t0, i0, bmaxi)
        v1 = cand_v[w, 1]
        i1 = cand_i[w, 1]
        t1 = (v1 < bminv) | ((v1 == bminv) & (i1 < bmini))
        bminv = jnp.where(t1, v1, bminv)
        bmini = jnp.where(t1, i1, bmini)
    cp0 = pltpu.make_async_copy(feats.at[pl.ds(bmaxi, 1), :],
                                out.at[pl.ds(0, 1), :], sem0)
    cp1 = pltpu.make_async_copy(feats.at[pl.ds(bmini, 1), :],
                                out.at[pl.ds(1, 1), :], sem1)
    cp0.start()
    cp1.start()
    cp0.wait()
    cp1.wait()


_merge = pl.pallas_call(
    _merge_body,
    in_specs=[
        pl.BlockSpec(memory_space=pltpu.SMEM),
        pl.BlockSpec(memory_space=pltpu.SMEM),
        pl.BlockSpec(memory_space=pl.ANY),
    ],
    out_specs=pl.BlockSpec(memory_space=pltpu.VMEM),
    out_shape=jax.ShapeDtypeStruct((2, D), jnp.float32),
    scratch_shapes=[pltpu.SemaphoreType.DMA, pltpu.SemaphoreType.DMA],
)


def kernel(feats, W, b):
    del b
    wd = W[0] - W[1]
    vals, idxs = _scan_kernel(feats, wd)
    return _merge(vals, idxs, feats)
